# baseline (speedup 1.0000x reference)
import jax
import jax.numpy as jnp
from jax.experimental import pallas as pl
from jax.experimental.pallas import tpu as pltpu
from jax.experimental.pallas import tpu_sc as plsc

N_TOK = 65536
N_EMB = 1024
DIM = 64
BLK = 1024
NCHUNK = 2

SC_NC = 2
SC_NS = 16
SC_NW = SC_NC * SC_NS
SC_CH = 128


def _tc_body(x_ref, w_ref, idx_ref, loss_ref, acc_ref):
    i = pl.program_id(0)
    x = x_ref[...]
    w = w_ref[...]
    c = jax.lax.dot_general(x, w, (((1,), (1,)), ((), ())),
                            preferred_element_type=jnp.float32)
    a = jnp.sum(x * x, axis=1, keepdims=True)
    b = jnp.sum(w * w, axis=1)[None, :]
    dist = (a + b) - 2.0 * c
    m = jnp.min(dist, axis=1, keepdims=True)
    jidx = jax.lax.broadcasted_iota(
        jnp.int32, dist.shape, 1).astype(jnp.float32)
    idxf = jnp.min(jnp.where(dist == m, jidx, float(N_EMB)), axis=1)
    idx_ref[...] = idxf.astype(jnp.int32).reshape(BLK // 128, 128)

    @pl.when(i == 0)
    def _():
        acc_ref[0] = 0.0

    acc_ref[0] += jnp.sum(m)

    @pl.when(i == pl.num_programs(0) - 1)
    def _():
        loss_ref[...] = jnp.full((1, 1), acc_ref[0], dtype=jnp.float32)


def _tc_argmin_loss(x_chunk, weight):
    n = x_chunk.shape[0]
    return pl.pallas_call(
        _tc_body,
        grid=(n // BLK,),
        in_specs=[
            pl.BlockSpec((BLK, DIM), lambda i: (i, 0)),
            pl.BlockSpec((N_EMB, DIM), lambda i: (0, 0)),
        ],
        out_specs=[
            pl.BlockSpec((BLK // 128, 128), lambda i: (i, 0)),
            pl.BlockSpec((1, 1), lambda i: (0, 0)),
        ],
        out_shape=[
            jax.ShapeDtypeStruct((n // 128, 128), jnp.int32),
            jax.ShapeDtypeStruct((1, 1), jnp.float32),
        ],
        scratch_shapes=[pltpu.SMEM((1,), jnp.float32)],
    )(x_chunk, weight)


def _sc_gather(w_pad, idx):
    n = idx.shape[0]
    rows_per_w = n // SC_NW
    n_ch = rows_per_w // SC_CH
    mesh = plsc.VectorSubcoreMesh(core_axis_name="c", subcore_axis_name="s")

    @pl.kernel(out_type=jax.ShapeDtypeStruct((n, 128), jnp.float32),
               mesh=mesh,
               scratch_types=[
                   pltpu.VMEM((SC_CH,), jnp.int32),
                   pltpu.VMEM((SC_CH, 128), jnp.float32),
                   pltpu.SemaphoreType.DMA,
               ])
    def k(w_hbm, i_hbm, o_hbm, idx_v, rows_v, sem):
        wid = jax.lax.axis_index("s") * SC_NC + jax.lax.axis_index("c")
        base = wid * rows_per_w

        @pl.loop(0, n_ch)
        def _(c):
            off = base + c * SC_CH
            pltpu.sync_copy(i_hbm.at[pl.ds(off, SC_CH)], idx_v)
            pltpu.async_copy(w_hbm.at[idx_v], rows_v, sem).wait()
            pltpu.sync_copy(rows_v, o_hbm.at[pl.ds(off, SC_CH)])

    return k(w_pad, idx)


def kernel(inputs, weight):
    w_pad = jnp.concatenate(
        [weight, jnp.zeros((N_EMB, 128 - DIM), jnp.float32)], axis=1)
    ch = N_TOK // NCHUNK
    idx_parts, q_parts, loss_parts = [], [], []
    for k in range(NCHUNK):
        idx2d, lsum = _tc_argmin_loss(
            jax.lax.slice_in_dim(inputs, k * ch, (k + 1) * ch, axis=0), weight)
        idx = idx2d.reshape(ch)
        q_parts.append(_sc_gather(w_pad, idx)[:, :DIM])
        idx_parts.append(idx)
        loss_parts.append(lsum[0, 0])
    loss = sum(loss_parts) * (1.25 / (N_TOK * DIM))
    quantized = jnp.concatenate(q_parts, axis=0)
    indices = jnp.concatenate(idx_parts, axis=0)
    return loss, quantized, indices


# speedup vs baseline: 1.0627x; 1.0627x over previous
import jax
import jax.numpy as jnp
from jax.experimental import pallas as pl
from jax.experimental.pallas import tpu as pltpu
from jax.experimental.pallas import tpu_sc as plsc

N_TOK = 65536
N_EMB = 1024
DIM = 64
BLK = 1024
NCHUNK = 2
CHT = N_TOK // NCHUNK

SC_NC = 2
SC_NS = 16
SC_NW = SC_NC * SC_NS
SC_CH = 128

SL_BLK = 4096


def _tc_body(x_ref, w_ref, idx_ref, loss_ref, acc_ref):
    i = pl.program_id(0)
    x = x_ref[...]
    w = w_ref[...]
    c = jax.lax.dot_general(x, w, (((1,), (1,)), ((), ())),
                            preferred_element_type=jnp.float32)
    a = jnp.sum(x * x, axis=1, keepdims=True)
    b = jnp.sum(w * w, axis=1)[None, :]
    dist = (a + b) - 2.0 * c
    m = jnp.min(dist, axis=1, keepdims=True)
    jidx = jax.lax.broadcasted_iota(
        jnp.int32, dist.shape, 1).astype(jnp.float32)
    idxf = jnp.min(jnp.where(dist == m, jidx, float(N_EMB)), axis=1)
    idx_ref[...] = idxf.astype(jnp.int32).reshape(BLK // 128, 128)

    @pl.when(i == 0)
    def _():
        acc_ref[0] = 0.0

    acc_ref[0] += jnp.sum(m)

    @pl.when(i == pl.num_programs(0) - 1)
    def _():
        loss_ref[...] = jnp.full((1, 1), acc_ref[0], dtype=jnp.float32)


def _tc_argmin_loss(inputs, weight, k):
    g = CHT // BLK
    return pl.pallas_call(
        _tc_body,
        grid=(g,),
        in_specs=[
            pl.BlockSpec((BLK, DIM), lambda i, k=k: (i + k * g, 0)),
            pl.BlockSpec((N_EMB, DIM), lambda i: (0, 0)),
        ],
        out_specs=[
            pl.BlockSpec((BLK // 128, 128), lambda i: (i, 0)),
            pl.BlockSpec((1, 1), lambda i: (0, 0)),
        ],
        out_shape=[
            jax.ShapeDtypeStruct((CHT // 128, 128), jnp.int32),
            jax.ShapeDtypeStruct((1, 1), jnp.float32),
        ],
        scratch_shapes=[pltpu.SMEM((1,), jnp.float32)],
    )(inputs, weight)


def _sc_gather(w_pad, idx):
    n = idx.shape[0]
    rows_per_w = n // SC_NW
    n_ch = rows_per_w // SC_CH
    mesh = plsc.VectorSubcoreMesh(core_axis_name="c", subcore_axis_name="s")

    @pl.kernel(out_type=jax.ShapeDtypeStruct((n, 128), jnp.float32),
               mesh=mesh,
               scratch_types=[
                   pltpu.VMEM((SC_CH,), jnp.int32),
                   pltpu.VMEM((SC_CH, 128), jnp.float32),
                   pltpu.SemaphoreType.DMA,
               ])
    def k(w_hbm, i_hbm, o_hbm, idx_v, rows_v, sem):
        wid = jax.lax.axis_index("s") * SC_NC + jax.lax.axis_index("c")
        base = wid * rows_per_w

        @pl.loop(0, n_ch)
        def _(c):
            off = base + c * SC_CH
            pltpu.sync_copy(i_hbm.at[pl.ds(off, SC_CH)], idx_v)
            pltpu.async_copy(w_hbm.at[idx_v], rows_v, sem).wait()
            pltpu.sync_copy(rows_v, o_hbm.at[pl.ds(off, SC_CH)])

    return k(w_pad, idx)


def _slice_body(_, q_ref, o_ref):
    o_ref[...] = q_ref[:, :DIM]


def _slice_first(q_raw):
    return pl.pallas_call(
        lambda q_ref, o_ref: _slice_body(None, q_ref, o_ref),
        grid=(CHT // SL_BLK,),
        in_specs=[pl.BlockSpec((SL_BLK, 128), lambda i: (i, 0))],
        out_specs=pl.BlockSpec((SL_BLK, DIM), lambda i: (i, 0)),
        out_shape=jax.ShapeDtypeStruct((N_TOK, DIM), jnp.float32),
    )(q_raw)


def _slice_into(buf, q_raw, k):
    g = CHT // SL_BLK
    return pl.pallas_call(
        _slice_body,
        grid=(g,),
        in_specs=[
            pl.BlockSpec(memory_space=pl.ANY),
            pl.BlockSpec((SL_BLK, 128), lambda i: (i, 0)),
        ],
        out_specs=pl.BlockSpec((SL_BLK, DIM), lambda i, k=k: (i + k * g, 0)),
        out_shape=jax.ShapeDtypeStruct((N_TOK, DIM), jnp.float32),
        input_output_aliases={0: 0},
    )(buf, q_raw)


def kernel(inputs, weight):
    w_pad = jnp.concatenate(
        [weight, jnp.zeros((N_EMB, 128 - DIM), jnp.float32)], axis=1)
    idx_parts, loss_parts, q_raws = [], [], []
    for k in range(NCHUNK):
        idx2d, lsum = _tc_argmin_loss(inputs, weight, k)
        idx = idx2d.reshape(CHT)
        q_raws.append(_sc_gather(w_pad, idx))
        idx_parts.append(idx)
        loss_parts.append(lsum[0, 0])
    quantized = _slice_first(q_raws[0])
    for k in range(1, NCHUNK):
        quantized = _slice_into(quantized, q_raws[k], k)
    loss = sum(loss_parts) * (1.25 / (N_TOK * DIM))
    indices = jnp.concatenate(idx_parts, axis=0)
    return loss, quantized, indices
